# baseline (speedup 1.0000x reference)
import functools

import jax
import jax.numpy as jnp
from jax.experimental import pallas as pl

B, DIM, HEADS, DS, H, W = 1, 192, 6, 2, 384, 384
CC = DIM // HEADS
HP, WP = H // DS, W // DS
N = HP * WP
TOPKS = (CC // 2, CC * 2 // 3, CC * 3 // 4, CC * 4 // 5)

F32 = jnp.float32


def _qkv_pool_kernel(x_ref, w_ref, o_ref):
    y = jax.lax.dot_general(w_ref[...], x_ref[0], (((1,), (0,)), ((), ())),
                            preferred_element_type=F32)

    @pl.when(pl.program_id(1) == 0)
    def _():
        o_ref[...] = y

    @pl.when(pl.program_id(1) > 0)
    def _():
        o_ref[...] = jnp.maximum(o_ref[...], y)


def _qkv_pool(xp, w_qkv, cb=4608):
    return pl.pallas_call(
        _qkv_pool_kernel,
        grid=(N // cb, 4),
        in_specs=[
            pl.BlockSpec((1, DIM, cb), lambda i, p: (p, 0, i)),
            pl.BlockSpec((3 * DIM, DIM), lambda i, p: (0, 0)),
        ],
        out_specs=pl.BlockSpec((3 * DIM, cb), lambda i, p: (0, i)),
        out_shape=jax.ShapeDtypeStruct((3 * DIM, N), F32),
    )(xp, w_qkv)


def _shift_rows(x, d):
    c, h, w = x.shape
    z = jnp.zeros((c, 1, w), x.dtype)
    if d == 1:
        return jnp.concatenate([x[:, 1:, :], z], axis=1)
    if d == -1:
        return jnp.concatenate([z, x[:, :-1, :]], axis=1)
    return x


def _shift_cols(x, d):
    c, h, w = x.shape
    z = jnp.zeros((c, h, 1), x.dtype)
    if d == 1:
        return jnp.concatenate([x[:, :, 1:], z], axis=2)
    if d == -1:
        return jnp.concatenate([z, x[:, :, :-1]], axis=2)
    return x


def _dw_kernel(x_ref, w_ref, o_ref):
    xb = x_ref[...]
    wb = w_ref[...]
    acc = jnp.zeros(xb.shape, F32)
    for di in range(3):
        for dj in range(3):
            t = _shift_cols(_shift_rows(xb, di - 1), dj - 1)
            acc = acc + wb[:, 3 * di + dj][:, None, None] * t
    o_ref[...] = acc


def _dw_conv(qkv, w_dw, cb=32):
    w2 = w_dw.reshape(3 * DIM, 9)
    return pl.pallas_call(
        _dw_kernel,
        grid=(3 * DIM // cb,),
        in_specs=[
            pl.BlockSpec((cb, HP, WP), lambda i: (i, 0, 0)),
            pl.BlockSpec((cb, 9), lambda i: (i, 0)),
        ],
        out_specs=pl.BlockSpec((cb, HP, WP), lambda i: (i, 0, 0)),
        out_shape=jax.ShapeDtypeStruct((3 * DIM, HP, WP), F32),
    )(qkv, w2)


def _gram_kernel(q_ref, k_ref, g_ref, qss_ref, kss_ref):
    qb = q_ref[...]
    kb = k_ref[...]
    g = jax.lax.dot_general(qb, kb, (((1,), (1,)), ((), ())),
                            preferred_element_type=F32)
    qss = jnp.sum(qb * qb, axis=1, keepdims=True)
    kss = jnp.sum(kb * kb, axis=1, keepdims=True)

    @pl.when(pl.program_id(0) == 0)
    def _():
        g_ref[...] = g
        qss_ref[...] = qss
        kss_ref[...] = kss

    @pl.when(pl.program_id(0) > 0)
    def _():
        g_ref[...] += g
        qss_ref[...] += qss
        kss_ref[...] += kss


def _gram(qkvf, cb=4608):
    return pl.pallas_call(
        _gram_kernel,
        grid=(N // cb,),
        in_specs=[
            pl.BlockSpec((DIM, cb), lambda i: (0, i)),
            pl.BlockSpec((DIM, cb), lambda i: (1, i)),
        ],
        out_specs=[
            pl.BlockSpec((DIM, DIM), lambda i: (0, 0)),
            pl.BlockSpec((DIM, 1), lambda i: (0, 0)),
            pl.BlockSpec((DIM, 1), lambda i: (0, 0)),
        ],
        out_shape=[
            jax.ShapeDtypeStruct((DIM, DIM), F32),
            jax.ShapeDtypeStruct((DIM, 1), F32),
            jax.ShapeDtypeStruct((DIM, 1), F32),
        ],
    )(qkvf, qkvf)


def _mask_kernel(g_ref, qss_ref, kss_ref, t_ref, w_ref, a_ref):
    g = g_ref[...]
    qn = jnp.maximum(jnp.sqrt(qss_ref[...]), 1e-12)
    kn = jnp.maximum(jnp.sqrt(kss_ref[...]), 1e-12)
    gn = g / qn / kn.reshape(1, DIM)
    tvec = t_ref[...]
    wvec = w_ref[...]

    blocks = [gn[h * CC:(h + 1) * CC, h * CC:(h + 1) * CC] * tvec[0, h]
              for h in range(HEADS)]
    a = jnp.stack(blocks, axis=0)

    ad = a[:, :, :, None]
    ae = a[:, :, None, :]
    d_idx = jax.lax.broadcasted_iota(jnp.int32, (HEADS, CC, CC, CC), 2)
    e_idx = jax.lax.broadcasted_iota(jnp.int32, (HEADS, CC, CC, CC), 3)
    beats = (ae > ad) | ((ae == ad) & (e_idx < d_idx))
    rank = jnp.sum(beats.astype(jnp.int32), axis=3)

    rowmax = jnp.max(a, axis=2, keepdims=True)
    ex = jnp.exp(a - rowmax)
    comb = jnp.zeros(a.shape, F32)
    for i, kk in enumerate(TOPKS):
        m = (rank < kk).astype(F32)
        me = ex * m
        z = jnp.sum(me, axis=2, keepdims=True)
        comb = comb + (wvec[0, i] / z) * me

    comb_rows = comb.reshape(DIM, CC)
    tiled = jnp.tile(comb_rows, (1, HEADS))
    r_idx = jax.lax.broadcasted_iota(jnp.int32, (DIM, DIM), 0)
    c_idx = jax.lax.broadcasted_iota(jnp.int32, (DIM, DIM), 1)
    inblock = (r_idx // CC) == (c_idx // CC)
    a_ref[...] = jnp.where(inblock, tiled, jnp.zeros((DIM, DIM), F32))


def _mask_combine(g, qss, kss, temperature, wts):
    t2 = temperature.reshape(1, HEADS)
    return pl.pallas_call(
        _mask_kernel,
        grid=(1,),
        in_specs=[
            pl.BlockSpec((DIM, DIM), lambda i: (0, 0)),
            pl.BlockSpec((DIM, 1), lambda i: (0, 0)),
            pl.BlockSpec((DIM, 1), lambda i: (0, 0)),
            pl.BlockSpec((1, HEADS), lambda i: (0, 0)),
            pl.BlockSpec((1, 4), lambda i: (0, 0)),
        ],
        out_specs=pl.BlockSpec((DIM, DIM), lambda i: (0, 0)),
        out_shape=jax.ShapeDtypeStruct((DIM, DIM), F32),
    )(g, qss, kss, t2, wts)


def _out_kernel(v_ref, a_ref, wp_ref, o_ref):
    o = jax.lax.dot_general(a_ref[...], v_ref[...], (((1,), (0,)), ((), ())),
                            preferred_element_type=F32)
    o = 0.5 * o * (1.0 + jax.lax.erf(o * 0.7071067811865476))
    o_ref[...] = jax.lax.dot_general(wp_ref[...], o, (((1,), (0,)), ((), ())),
                                     preferred_element_type=F32)


def _out_proj(qkvf, abd, w_proj, cb=4608):
    return pl.pallas_call(
        _out_kernel,
        grid=(N // cb,),
        in_specs=[
            pl.BlockSpec((DIM, cb), lambda i: (2, i)),
            pl.BlockSpec((DIM, DIM), lambda i: (0, 0)),
            pl.BlockSpec((DIM, DIM), lambda i: (0, 0)),
        ],
        out_specs=pl.BlockSpec((DIM, cb), lambda i: (0, i)),
        out_shape=jax.ShapeDtypeStruct((DIM, N), F32),
    )(qkvf, abd, w_proj)


def kernel(x, W_qkv, W_dw, W_proj, temperature, attn1, attn2, attn3, attn4):
    x3 = x.reshape(DIM, H, W)
    xp = x3.reshape(DIM, HP, 2, WP, 2).transpose(2, 4, 0, 1, 3)
    xp = xp.reshape(4, DIM, N)
    qkvf = _qkv_pool(xp, W_qkv)
    qkv3 = _dw_conv(qkvf.reshape(3 * DIM, HP, WP), W_dw)
    qkvf = qkv3.reshape(3 * DIM, N)
    g, qss, kss = _gram(qkvf)
    wts = jnp.concatenate([attn1, attn2, attn3, attn4]).reshape(1, 4)
    abd = _mask_combine(g, qss, kss, temperature, wts)
    zq = _out_proj(qkvf, abd, W_proj)
    z4 = jnp.broadcast_to(zq.reshape(DIM, HP, 1, WP, 1),
                          (DIM, HP, 2, WP, 2))
    return z4.reshape(B, DIM, H, W)


# speedup vs baseline: 1.7211x; 1.7211x over previous
import functools

import jax
import jax.numpy as jnp
from jax.experimental import pallas as pl

B, DIM, HEADS, DS, H, W = 1, 192, 6, 2, 384, 384
CC = DIM // HEADS
HP, WP = H // DS, W // DS
NU = HP * W
TOPKS = (CC // 2, CC * 2 // 3, CC * 3 // 4, CC * 4 // 5)

F32 = jnp.float32


def _shift_left(x, d):
    z = jnp.zeros(x.shape[:-1] + (d,), x.dtype)
    return jnp.concatenate([x[..., d:], z], axis=-1)


def _qkv_pool_kernel(x_ref, w_ref, o_ref, *, rbh):
    w = w_ref[...]
    par = jax.lax.broadcasted_iota(jnp.int32, (3 * DIM, W), 1) % 2
    for r in range(rbh):
        y0 = jax.lax.dot_general(w, x_ref[:, 2 * r, :],
                                 (((1,), (0,)), ((), ())),
                                 preferred_element_type=F32)
        y1 = jax.lax.dot_general(w, x_ref[:, 2 * r + 1, :],
                                 (((1,), (0,)), ((), ())),
                                 preferred_element_type=F32)
        m = jnp.maximum(y0, y1)
        cm = jnp.maximum(m, _shift_left(m, 1))
        o_ref[:, r, :] = jnp.where(par == 0, cm, jnp.zeros_like(cm))


def _qkv_pool(x3, w_qkv, rbh=8):
    return pl.pallas_call(
        functools.partial(_qkv_pool_kernel, rbh=rbh),
        grid=(HP // rbh,),
        in_specs=[
            pl.BlockSpec((DIM, 2 * rbh, W), lambda i: (0, i, 0)),
            pl.BlockSpec((3 * DIM, DIM), lambda i: (0, 0)),
        ],
        out_specs=pl.BlockSpec((3 * DIM, rbh, W), lambda i: (0, i, 0)),
        out_shape=jax.ShapeDtypeStruct((3 * DIM, HP, W), F32),
    )(x3, w_qkv)


def _shift_rows(x, d):
    c, h, w = x.shape
    z = jnp.zeros((c, 1, w), x.dtype)
    if d == 1:
        return jnp.concatenate([x[:, 1:, :], z], axis=1)
    if d == -1:
        return jnp.concatenate([z, x[:, :-1, :]], axis=1)
    return x


def _shift_cols2(x, d):
    c, h, w = x.shape
    z = jnp.zeros((c, h, 2), x.dtype)
    if d == 1:
        return jnp.concatenate([x[:, :, 2:], z], axis=2)
    if d == -1:
        return jnp.concatenate([z, x[:, :, :-2]], axis=2)
    return x


def _dw_kernel(x_ref, w_ref, o_ref):
    xb = x_ref[...]
    wb = w_ref[...]
    acc = jnp.zeros(xb.shape, F32)
    for di in range(3):
        for dj in range(3):
            t = _shift_cols2(_shift_rows(xb, di - 1), dj - 1)
            acc = acc + wb[:, 3 * di + dj][:, None, None] * t
    o_ref[...] = acc


def _dw_conv(qkv, w_dw, cb=16):
    w2 = w_dw.reshape(3 * DIM, 9)
    return pl.pallas_call(
        _dw_kernel,
        grid=(3 * DIM // cb,),
        in_specs=[
            pl.BlockSpec((cb, HP, W), lambda i: (i, 0, 0)),
            pl.BlockSpec((cb, 9), lambda i: (i, 0)),
        ],
        out_specs=pl.BlockSpec((cb, HP, W), lambda i: (i, 0, 0)),
        out_shape=jax.ShapeDtypeStruct((3 * DIM, HP, W), F32),
    )(qkv, w2)


def _gram_kernel(q_ref, k_ref, g_ref, qss_ref, kss_ref):
    qb = q_ref[...]
    kb = k_ref[...]
    g = jax.lax.dot_general(qb, kb, (((1,), (1,)), ((), ())),
                            preferred_element_type=F32)
    qss = jnp.sum(qb * qb, axis=1, keepdims=True)
    kss = jnp.sum(kb * kb, axis=1, keepdims=True)

    @pl.when(pl.program_id(0) == 0)
    def _():
        g_ref[...] = g
        qss_ref[...] = qss
        kss_ref[...] = kss

    @pl.when(pl.program_id(0) > 0)
    def _():
        g_ref[...] += g
        qss_ref[...] += qss
        kss_ref[...] += kss


def _gram(qkvf, cb=4608):
    return pl.pallas_call(
        _gram_kernel,
        grid=(NU // cb,),
        in_specs=[
            pl.BlockSpec((DIM, cb), lambda i: (0, i)),
            pl.BlockSpec((DIM, cb), lambda i: (1, i)),
        ],
        out_specs=[
            pl.BlockSpec((DIM, DIM), lambda i: (0, 0)),
            pl.BlockSpec((DIM, 1), lambda i: (0, 0)),
            pl.BlockSpec((DIM, 1), lambda i: (0, 0)),
        ],
        out_shape=[
            jax.ShapeDtypeStruct((DIM, DIM), F32),
            jax.ShapeDtypeStruct((DIM, 1), F32),
            jax.ShapeDtypeStruct((DIM, 1), F32),
        ],
    )(qkvf, qkvf)


def _mask_kernel(g_ref, qss_ref, kss_ref, t_ref, w_ref, a_ref):
    g = g_ref[...]
    qn = jnp.maximum(jnp.sqrt(qss_ref[...]), 1e-12)
    kn = jnp.maximum(jnp.sqrt(kss_ref[...]), 1e-12)
    gn = g / qn / kn.reshape(1, DIM)
    tvec = t_ref[...]
    wvec = w_ref[...]

    blocks = [gn[h * CC:(h + 1) * CC, h * CC:(h + 1) * CC] * tvec[0, h]
              for h in range(HEADS)]
    a = jnp.stack(blocks, axis=0)

    ad = a[:, :, :, None]
    ae = a[:, :, None, :]
    d_idx = jax.lax.broadcasted_iota(jnp.int32, (HEADS, CC, CC, CC), 2)
    e_idx = jax.lax.broadcasted_iota(jnp.int32, (HEADS, CC, CC, CC), 3)
    beats = (ae > ad) | ((ae == ad) & (e_idx < d_idx))
    rank = jnp.sum(beats.astype(jnp.int32), axis=3)

    rowmax = jnp.max(a, axis=2, keepdims=True)
    ex = jnp.exp(a - rowmax)
    comb = jnp.zeros(a.shape, F32)
    for i, kk in enumerate(TOPKS):
        m = (rank < kk).astype(F32)
        me = ex * m
        z = jnp.sum(me, axis=2, keepdims=True)
        comb = comb + (wvec[0, i] / z) * me

    comb_rows = comb.reshape(DIM, CC)
    tiled = jnp.tile(comb_rows, (1, HEADS))
    r_idx = jax.lax.broadcasted_iota(jnp.int32, (DIM, DIM), 0)
    c_idx = jax.lax.broadcasted_iota(jnp.int32, (DIM, DIM), 1)
    inblock = (r_idx // CC) == (c_idx // CC)
    a_ref[...] = jnp.where(inblock, tiled, jnp.zeros((DIM, DIM), F32))


def _mask_combine(g, qss, kss, temperature, wts):
    t2 = temperature.reshape(1, HEADS)
    return pl.pallas_call(
        _mask_kernel,
        grid=(1,),
        in_specs=[
            pl.BlockSpec((DIM, DIM), lambda i: (0, 0)),
            pl.BlockSpec((DIM, 1), lambda i: (0, 0)),
            pl.BlockSpec((DIM, 1), lambda i: (0, 0)),
            pl.BlockSpec((1, HEADS), lambda i: (0, 0)),
            pl.BlockSpec((1, 4), lambda i: (0, 0)),
        ],
        out_specs=pl.BlockSpec((DIM, DIM), lambda i: (0, 0)),
        out_shape=jax.ShapeDtypeStruct((DIM, DIM), F32),
    )(g, qss, kss, t2, wts)


def _out_kernel(v_ref, a_ref, wp_ref, o_ref):
    o = jax.lax.dot_general(a_ref[...], v_ref[...], (((1,), (0,)), ((), ())),
                            preferred_element_type=F32)
    o = 0.5 * o * (1.0 + jax.lax.erf(o * 0.7071067811865476))
    z = jax.lax.dot_general(wp_ref[...], o, (((1,), (0,)), ((), ())),
                            preferred_element_type=F32)
    zr = jnp.concatenate([jnp.zeros((DIM, 1), F32), z[:, :-1]], axis=1)
    o_ref[...] = z + zr


def _out_proj(qkvf, abd, w_proj, cb=4608):
    return pl.pallas_call(
        _out_kernel,
        grid=(NU // cb,),
        in_specs=[
            pl.BlockSpec((DIM, cb), lambda i: (2, i)),
            pl.BlockSpec((DIM, DIM), lambda i: (0, 0)),
            pl.BlockSpec((DIM, DIM), lambda i: (0, 0)),
        ],
        out_specs=pl.BlockSpec((DIM, cb), lambda i: (0, i)),
        out_shape=jax.ShapeDtypeStruct((DIM, NU), F32),
    )(qkvf, abd, w_proj)


def kernel(x, W_qkv, W_dw, W_proj, temperature, attn1, attn2, attn3, attn4):
    x3 = x.reshape(DIM, H, W)
    qkv3 = _qkv_pool(x3, W_qkv)
    qkv3 = _dw_conv(qkv3, W_dw)
    qkvf = qkv3.reshape(3 * DIM, NU)
    g, qss, kss = _gram(qkvf)
    wts = jnp.concatenate([attn1, attn2, attn3, attn4]).reshape(1, 4)
    abd = _mask_combine(g, qss, kss, temperature, wts)
    z = _out_proj(qkvf, abd, W_proj)
    z4 = jnp.broadcast_to(z.reshape(DIM, HP, 1, W), (DIM, HP, 2, W))
    return z4.reshape(B, DIM, H, W)


# speedup vs baseline: 1.9830x; 1.1521x over previous
import functools

import jax
import jax.numpy as jnp
from jax.experimental import pallas as pl

B, DIM, HEADS, DS, H, W = 1, 192, 6, 2, 384, 384
CC = DIM // HEADS
HP, WP = H // DS, W // DS
NU = HP * W
TOPKS = (CC // 2, CC * 2 // 3, CC * 3 // 4, CC * 4 // 5)

F32 = jnp.float32


def _shift_left(x, d):
    z = jnp.zeros(x.shape[:-1] + (d,), x.dtype)
    return jnp.concatenate([x[..., d:], z], axis=-1)


def _qkv_pool_kernel(x_ref, w_ref, o_ref, *, rbh):
    w = w_ref[...]
    par = jax.lax.broadcasted_iota(jnp.int32, (3 * DIM, W), 1) % 2
    for r in range(rbh):
        y0 = jax.lax.dot_general(w, x_ref[:, 2 * r, :],
                                 (((1,), (0,)), ((), ())),
                                 preferred_element_type=F32)
        y1 = jax.lax.dot_general(w, x_ref[:, 2 * r + 1, :],
                                 (((1,), (0,)), ((), ())),
                                 preferred_element_type=F32)
        m = jnp.maximum(y0, y1)
        cm = jnp.maximum(m, _shift_left(m, 1))
        o_ref[:, r, :] = jnp.where(par == 0, cm, jnp.zeros_like(cm))


def _qkv_pool(x3, w_qkv, rbh=8):
    return pl.pallas_call(
        functools.partial(_qkv_pool_kernel, rbh=rbh),
        grid=(HP // rbh,),
        in_specs=[
            pl.BlockSpec((DIM, 2 * rbh, W), lambda i: (0, i, 0)),
            pl.BlockSpec((3 * DIM, DIM), lambda i: (0, 0)),
        ],
        out_specs=pl.BlockSpec((3 * DIM, rbh, W), lambda i: (0, i, 0)),
        out_shape=jax.ShapeDtypeStruct((3 * DIM, HP, W), F32),
    )(x3, w_qkv)


def _shift_rows(x, d):
    c, h, w = x.shape
    z = jnp.zeros((c, 1, w), x.dtype)
    if d == 1:
        return jnp.concatenate([x[:, 1:, :], z], axis=1)
    if d == -1:
        return jnp.concatenate([z, x[:, :-1, :]], axis=1)
    return x


def _shift_cols2(x, d):
    c, h, w = x.shape
    z = jnp.zeros((c, h, 2), x.dtype)
    if d == 1:
        return jnp.concatenate([x[:, :, 2:], z], axis=2)
    if d == -1:
        return jnp.concatenate([z, x[:, :, :-2]], axis=2)
    return x


def _dw_kernel(x_ref, w_ref, o_ref):
    xb = x_ref[...]
    wb = w_ref[...]
    acc = jnp.zeros(xb.shape, F32)
    for di in range(3):
        for dj in range(3):
            t = _shift_cols2(_shift_rows(xb, di - 1), dj - 1)
            acc = acc + wb[:, 3 * di + dj][:, None, None] * t
    o_ref[...] = acc


def _dw_conv(qkv, w_dw, cb=16):
    w2 = w_dw.reshape(3 * DIM, 9)
    return pl.pallas_call(
        _dw_kernel,
        grid=(3 * DIM // cb,),
        in_specs=[
            pl.BlockSpec((cb, HP, W), lambda i: (i, 0, 0)),
            pl.BlockSpec((cb, 9), lambda i: (i, 0)),
        ],
        out_specs=pl.BlockSpec((cb, HP, W), lambda i: (i, 0, 0)),
        out_shape=jax.ShapeDtypeStruct((3 * DIM, HP, W), F32),
    )(qkv, w2)


def _gram_kernel(q_ref, k_ref, g_ref, qss_ref, kss_ref, *, rb):
    g = jnp.zeros((DIM, DIM), F32)
    qss = jnp.zeros((DIM, 1), F32)
    kss = jnp.zeros((DIM, 1), F32)
    for r in range(rb):
        qb = q_ref[:, r, :]
        kb = k_ref[:, r, :]
        g = g + jax.lax.dot_general(qb, kb, (((1,), (1,)), ((), ())),
                                    preferred_element_type=F32)
        qss = qss + jnp.sum(qb * qb, axis=1, keepdims=True)
        kss = kss + jnp.sum(kb * kb, axis=1, keepdims=True)

    @pl.when(pl.program_id(0) == 0)
    def _():
        g_ref[...] = g
        qss_ref[...] = qss
        kss_ref[...] = kss

    @pl.when(pl.program_id(0) > 0)
    def _():
        g_ref[...] += g
        qss_ref[...] += qss
        kss_ref[...] += kss


def _gram(qkv3, rb=16):
    return pl.pallas_call(
        functools.partial(_gram_kernel, rb=rb),
        grid=(HP // rb,),
        in_specs=[
            pl.BlockSpec((DIM, rb, W), lambda i: (0, i, 0)),
            pl.BlockSpec((DIM, rb, W), lambda i: (1, i, 0)),
        ],
        out_specs=[
            pl.BlockSpec((DIM, DIM), lambda i: (0, 0)),
            pl.BlockSpec((DIM, 1), lambda i: (0, 0)),
            pl.BlockSpec((DIM, 1), lambda i: (0, 0)),
        ],
        out_shape=[
            jax.ShapeDtypeStruct((DIM, DIM), F32),
            jax.ShapeDtypeStruct((DIM, 1), F32),
            jax.ShapeDtypeStruct((DIM, 1), F32),
        ],
    )(qkv3, qkv3)


def _mask_kernel(g_ref, qss_ref, kss_ref, t_ref, w_ref, a_ref):
    g = g_ref[...]
    qn = jnp.maximum(jnp.sqrt(qss_ref[...]), 1e-12)
    kn = jnp.maximum(jnp.sqrt(kss_ref[...]), 1e-12)
    gn = g / qn / kn.reshape(1, DIM)
    tvec = t_ref[...]
    wvec = w_ref[...]

    blocks = [gn[h * CC:(h + 1) * CC, h * CC:(h + 1) * CC] * tvec[0, h]
              for h in range(HEADS)]
    a = jnp.stack(blocks, axis=0)

    ad = a[:, :, :, None]
    ae = a[:, :, None, :]
    d_idx = jax.lax.broadcasted_iota(jnp.int32, (HEADS, CC, CC, CC), 2)
    e_idx = jax.lax.broadcasted_iota(jnp.int32, (HEADS, CC, CC, CC), 3)
    beats = (ae > ad) | ((ae == ad) & (e_idx < d_idx))
    rank = jnp.sum(beats.astype(jnp.int32), axis=3)

    rowmax = jnp.max(a, axis=2, keepdims=True)
    ex = jnp.exp(a - rowmax)
    comb = jnp.zeros(a.shape, F32)
    for i, kk in enumerate(TOPKS):
        m = (rank < kk).astype(F32)
        me = ex * m
        z = jnp.sum(me, axis=2, keepdims=True)
        comb = comb + (wvec[0, i] / z) * me

    comb_rows = comb.reshape(DIM, CC)
    tiled = jnp.tile(comb_rows, (1, HEADS))
    r_idx = jax.lax.broadcasted_iota(jnp.int32, (DIM, DIM), 0)
    c_idx = jax.lax.broadcasted_iota(jnp.int32, (DIM, DIM), 1)
    inblock = (r_idx // CC) == (c_idx // CC)
    a_ref[...] = jnp.where(inblock, tiled, jnp.zeros((DIM, DIM), F32))


def _mask_combine(g, qss, kss, temperature, wts):
    t2 = temperature.reshape(1, HEADS)
    return pl.pallas_call(
        _mask_kernel,
        grid=(1,),
        in_specs=[
            pl.BlockSpec((DIM, DIM), lambda i: (0, 0)),
            pl.BlockSpec((DIM, 1), lambda i: (0, 0)),
            pl.BlockSpec((DIM, 1), lambda i: (0, 0)),
            pl.BlockSpec((1, HEADS), lambda i: (0, 0)),
            pl.BlockSpec((1, 4), lambda i: (0, 0)),
        ],
        out_specs=pl.BlockSpec((DIM, DIM), lambda i: (0, 0)),
        out_shape=jax.ShapeDtypeStruct((DIM, DIM), F32),
    )(g, qss, kss, t2, wts)


def _out_kernel(v_ref, a_ref, wp_ref, o_ref, *, rb):
    av = a_ref[...]
    wp = wp_ref[...]
    for r in range(rb):
        o = jax.lax.dot_general(av, v_ref[:, r, :], (((1,), (0,)), ((), ())),
                                preferred_element_type=F32)
        o = 0.5 * o * (1.0 + jax.lax.erf(o * 0.7071067811865476))
        z = jax.lax.dot_general(wp, o, (((1,), (0,)), ((), ())),
                                preferred_element_type=F32)
        zr = jnp.concatenate([jnp.zeros((DIM, 1), F32), z[:, :-1]], axis=1)
        zz = z + zr
        o_ref[:, r, 0, :] = zz
        o_ref[:, r, 1, :] = zz


def _out_proj(qkv3, abd, w_proj, rb=8):
    return pl.pallas_call(
        functools.partial(_out_kernel, rb=rb),
        grid=(HP // rb,),
        in_specs=[
            pl.BlockSpec((DIM, rb, W), lambda i: (2, i, 0)),
            pl.BlockSpec((DIM, DIM), lambda i: (0, 0)),
            pl.BlockSpec((DIM, DIM), lambda i: (0, 0)),
        ],
        out_specs=pl.BlockSpec((DIM, rb, 2, W), lambda i: (0, i, 0, 0)),
        out_shape=jax.ShapeDtypeStruct((DIM, HP, 2, W), F32),
    )(qkv3, abd, w_proj)


def kernel(x, W_qkv, W_dw, W_proj, temperature, attn1, attn2, attn3, attn4):
    x3 = x.reshape(DIM, H, W)
    qkv3 = _qkv_pool(x3, W_qkv)
    qkv3 = _dw_conv(qkv3, W_dw)
    g, qss, kss = _gram(qkv3)
    wts = jnp.concatenate([attn1, attn2, attn3, attn4]).reshape(1, 4)
    abd = _mask_combine(g, qss, kss, temperature, wts)
    z4 = _out_proj(qkv3, abd, W_proj)
    return z4.reshape(B, DIM, H, W)
